# baseline (speedup 1.0000x reference)
import functools

import jax
import jax.numpy as jnp
from jax.experimental import pallas as pl
from jax.experimental.pallas import tpu as pltpu


def _taps_nhwc(x, w_ref, oh, ow, c):
    n = x.shape[0]
    m = n * oh * ow
    a = jnp.concatenate(
        [x[:, kh:kh + oh, kw:kw + ow, :].reshape(m, c)
         for kh in range(3) for kw in range(3)], axis=1)
    return jnp.dot(a, w_ref[...], preferred_element_type=jnp.float32)


def _taps_hwnc(x, w_ref, oh, ow, c):
    n = x.shape[2]
    m = oh * ow * n
    a = jnp.concatenate(
        [x[kh:kh + oh, kw:kw + ow, :, :].reshape(m, c)
         for kh in range(3) for kw in range(3)], axis=1)
    return jnp.dot(a, w_ref[...], preferred_element_type=jnp.float32)


def _conv_batch_kernel(x_ref, w_ref, b_ref, o_ref, *, oh, ow, c):
    n = x_ref.shape[0]
    acc = _taps_nhwc(x_ref[...], w_ref, oh, ow, c)
    r = jnp.maximum(acc + b_ref[...], 0.0)
    o_ref[...] = r.reshape(n, oh, ow, o_ref.shape[-1]).astype(o_ref.dtype)


def _conv_hwnc_kernel(x_ref, w_ref, b_ref, o_ref, *, oh, ow, c,
                      transpose_in):
    x = x_ref[...]
    if transpose_in:
        x = jnp.transpose(x, (1, 2, 0, 3))
    n = x.shape[2]
    acc = _taps_hwnc(x, w_ref, oh, ow, c)
    r = jnp.maximum(acc + b_ref[...], 0.0)
    o_ref[...] = r.reshape(oh, ow, n,
                           o_ref.shape[-1]).astype(jnp.bfloat16)


def _tail_kernel(x_ref, w_ref, b_ref, dlw_ref, o_ref, *, c):
    x = x_ref[...]
    n = x.shape[2]
    tn = w_ref.shape[1]
    acc = _taps_hwnc(x, w_ref, 2, 2, c)
    r = jnp.maximum(acc + b_ref[...], 0.0).astype(jnp.bfloat16)
    pooled = jnp.mean(r.reshape(4, n, tn).astype(jnp.float32), axis=0)
    h_part = jnp.dot(pooled.astype(jnp.bfloat16), dlw_ref[...],
                     preferred_element_type=jnp.float32)
    o_ref[...] = h_part.reshape(o_ref.shape)


def _head_kernel(hp_ref, dlb_ref, fcw_ref, fcb_ref, o_ref):
    h = hp_ref[0] + hp_ref[1] + dlb_ref[...]
    logits = jnp.dot(h.astype(jnp.bfloat16), fcw_ref[...],
                     preferred_element_type=jnp.float32) + fcb_ref[...]
    o_ref[...] = logits


def _vmem_limit(*arrays):
    need = 2 * sum(a.size * a.dtype.itemsize for a in arrays) + (8 << 20)
    return int(min(max(need, 32 << 20), 58 << 20))


def _conv_batch(x, w, b):
    n, h, wd, c = x.shape
    cout = w.shape[1]
    oh, ow = h - 2, wd - 2
    nb = n // 2
    return pl.pallas_call(
        functools.partial(_conv_batch_kernel, oh=oh, ow=ow, c=c),
        out_shape=jax.ShapeDtypeStruct((n, oh, ow, cout), jnp.bfloat16),
        grid=(2,),
        in_specs=[
            pl.BlockSpec((nb, h, wd, c), lambda i: (i, 0, 0, 0)),
            pl.BlockSpec(w.shape, lambda i: (0, 0)),
            pl.BlockSpec((1, cout), lambda i: (0, 0)),
        ],
        out_specs=pl.BlockSpec((nb, oh, ow, cout), lambda i: (i, 0, 0, 0)),
        compiler_params=pltpu.CompilerParams(
            dimension_semantics=("parallel",),
            vmem_limit_bytes=_vmem_limit(x, w, b)),
    )(x, w, b)


def _conv_hwnc(x, w, b, *, transpose_in=False):
    if transpose_in:
        n, h, wd, c = x.shape
    else:
        h, wd, n, c = x.shape
    cout = w.shape[1]
    oh, ow = h - 2, wd - 2
    tn = cout // 2
    return pl.pallas_call(
        functools.partial(_conv_hwnc_kernel, oh=oh, ow=ow, c=c,
                          transpose_in=transpose_in),
        out_shape=jax.ShapeDtypeStruct((oh, ow, n, cout), jnp.bfloat16),
        grid=(2,),
        in_specs=[
            pl.BlockSpec(x.shape, lambda i: (0, 0, 0, 0)),
            pl.BlockSpec((w.shape[0], tn), lambda i: (0, i)),
            pl.BlockSpec((1, tn), lambda i: (0, i)),
        ],
        out_specs=pl.BlockSpec((oh, ow, n, tn), lambda i: (0, 0, 0, i)),
        compiler_params=pltpu.CompilerParams(
            dimension_semantics=("parallel",),
            vmem_limit_bytes=_vmem_limit(x, w, b)),
    )(x, w, b)


def _tail(x, w, b, dl_w, dl_b, fc_w, fc_b):
    h, wd, n, c = x.shape
    cout = w.shape[1]
    tn = cout // 2
    nh = dl_w.shape[1]
    h_parts = pl.pallas_call(
        functools.partial(_tail_kernel, c=c),
        out_shape=jax.ShapeDtypeStruct((2, n, nh), jnp.float32),
        grid=(2,),
        in_specs=[
            pl.BlockSpec(x.shape, lambda i: (0, 0, 0, 0)),
            pl.BlockSpec((w.shape[0], tn), lambda i: (0, i)),
            pl.BlockSpec((1, tn), lambda i: (0, i)),
            pl.BlockSpec((tn, nh), lambda i: (i, 0)),
        ],
        out_specs=pl.BlockSpec((1, n, nh), lambda i: (i, 0, 0)),
        compiler_params=pltpu.CompilerParams(
            dimension_semantics=("parallel",),
            vmem_limit_bytes=_vmem_limit(x, w, dl_w)),
    )(x, w, b, dl_w)
    logits = pl.pallas_call(
        _head_kernel,
        out_shape=jax.ShapeDtypeStruct((n, fc_w.shape[1]), jnp.float32),
        in_specs=[pl.BlockSpec(memory_space=pltpu.MemorySpace.VMEM)] * 4,
        out_specs=pl.BlockSpec(memory_space=pltpu.MemorySpace.VMEM),
    )(h_parts, dl_b, fc_w, fc_b)
    return logits


def kernel(x, conv1_w, conv1_b, conv2_w, conv2_b, conv3_w, conv3_b,
           conv4_w, conv4_b, conv5_w, conv5_b, conv6_w, conv6_b,
           conv7_w, conv7_b, conv8_w, conv8_b, dl_w, dl_b, fc_w, fc_b):
    xh = jnp.transpose(x, (0, 2, 3, 1)).astype(jnp.bfloat16)
    cin = xh.shape[-1]
    cpad = 384
    xh = jnp.pad(xh, ((0, 0), (0, 0), (0, 0), (0, cpad - cin)))
    w1 = conv1_w[:9 * cin].reshape(9, cin, conv1_w.shape[1])
    w1 = jnp.pad(w1, ((0, 0), (0, cpad - cin), (0, 0)))
    w1 = w1.reshape(9 * cpad, conv1_w.shape[1])

    h = _conv_batch(xh, w1, conv1_b)
    h = _conv_hwnc(h, conv2_w, conv2_b, transpose_in=True)
    h = _conv_hwnc(h, conv3_w, conv3_b)
    h = _conv_hwnc(h, conv4_w, conv4_b)
    h = _conv_hwnc(h, conv5_w, conv5_b)
    h = _conv_hwnc(h, conv6_w, conv6_b)
    h = _conv_hwnc(h, conv7_w, conv7_b)
    logits = _tail(h, conv8_w, conv8_b, dl_w, dl_b, fc_w, fc_b)
    return logits[:, :2]


# speedup vs baseline: 1.1152x; 1.1152x over previous
import functools

import jax
import jax.numpy as jnp
from jax.experimental import pallas as pl
from jax.experimental.pallas import tpu as pltpu


def _taps_nhwc(x, w_ref, oh, ow, c):
    n = x.shape[0]
    m = n * oh * ow
    a = jnp.concatenate(
        [x[:, kh:kh + oh, kw:kw + ow, :].reshape(m, c)
         for kh in range(3) for kw in range(3)], axis=1)
    return jnp.dot(a, w_ref[...], preferred_element_type=jnp.float32)


def _taps_hwnc(x, w_ref, oh, ow, c):
    n = x.shape[2]
    m = oh * ow * n
    a = jnp.concatenate(
        [x[kh:kh + oh, kw:kw + ow, :, :].reshape(m, c)
         for kh in range(3) for kw in range(3)], axis=1)
    return jnp.dot(a, w_ref[...], preferred_element_type=jnp.float32)


def _conv1_kernel(x_ref, w_ref, b_ref, o_ref, *, oh, ow, c):
    n = x_ref.shape[0]
    m = n * oh * ow
    x = x_ref[...]
    kw_pad = w_ref.shape[0]
    pieces = [x[:, kh:kh + oh, kw:kw + ow, :].reshape(m, c)
              for kh in range(3) for kw in range(3)]
    pieces.append(jnp.zeros((m, kw_pad - 9 * c), jnp.bfloat16))
    a = jnp.concatenate(pieces, axis=1)
    r = jnp.dot(a, w_ref[...], preferred_element_type=jnp.float32)
    r = jnp.maximum(r + b_ref[...], 0.0)
    o_ref[...] = r.reshape(n, oh, ow, o_ref.shape[-1]).astype(o_ref.dtype)


def _conv_batch_kernel(x_ref, w_ref, b_ref, o_ref, *, oh, ow, c):
    n = x_ref.shape[0]
    acc = _taps_nhwc(x_ref[...], w_ref, oh, ow, c)
    r = jnp.maximum(acc + b_ref[...], 0.0)
    o_ref[...] = r.reshape(n, oh, ow, o_ref.shape[-1]).astype(o_ref.dtype)


def _conv_hwnc_kernel(x_ref, w_ref, b_ref, o_ref, *, oh, ow, c,
                      transpose_in):
    x = x_ref[...]
    if transpose_in:
        x = jnp.transpose(x, (1, 2, 0, 3))
    x = x.astype(jnp.bfloat16)
    n = x.shape[2]
    acc = _taps_hwnc(x, w_ref, oh, ow, c)
    r = jnp.maximum(acc + b_ref[...], 0.0)
    r = r.astype(jnp.bfloat16).astype(jnp.float32)
    o_ref[...] = r.reshape(oh, ow, n, o_ref.shape[-1])


def _tail_kernel(x_ref, w_ref, b_ref, dlw_ref, o_ref, *, c):
    x = x_ref[...]
    n = x.shape[0]
    tn = w_ref.shape[1]
    acc = _taps_nhwc(x, w_ref, 2, 2, c)
    r = jnp.maximum(acc + b_ref[...], 0.0).astype(jnp.bfloat16)
    pooled = jnp.mean(r.reshape(n, 4, tn).astype(jnp.float32), axis=1)
    h_part = jnp.dot(pooled.astype(jnp.bfloat16), dlw_ref[...],
                     preferred_element_type=jnp.float32)
    o_ref[...] = h_part.reshape(o_ref.shape)


def _head_kernel(hp_ref, dlb_ref, fcw_ref, fcb_ref, o_ref):
    h = hp_ref[0] + hp_ref[1] + dlb_ref[...]
    logits = jnp.dot(h.astype(jnp.bfloat16), fcw_ref[...],
                     preferred_element_type=jnp.float32) + fcb_ref[...]
    o_ref[...] = logits


def _vmem_limit(*arrays):
    need = 2 * sum(a.size * a.dtype.itemsize for a in arrays) + (8 << 20)
    return int(min(max(need, 32 << 20), 58 << 20))


def _conv_single(x, w, b, kern_fn):
    n, h, wd, c = x.shape
    cout = w.shape[1]
    oh, ow = h - 2, wd - 2
    return pl.pallas_call(
        functools.partial(kern_fn, oh=oh, ow=ow, c=c),
        out_shape=jax.ShapeDtypeStruct((n, oh, ow, cout), jnp.bfloat16),
        in_specs=[pl.BlockSpec(memory_space=pltpu.MemorySpace.VMEM)] * 3,
        out_specs=pl.BlockSpec(memory_space=pltpu.MemorySpace.VMEM),
        compiler_params=pltpu.CompilerParams(
            vmem_limit_bytes=_vmem_limit(x, w, b)),
    )(x, w, b)


def _conv_csplit(x, w, b):
    n, h, wd, c = x.shape
    cout = w.shape[1]
    oh, ow = h - 2, wd - 2
    tn = cout // 2
    return pl.pallas_call(
        functools.partial(_conv_batch_kernel, oh=oh, ow=ow, c=c),
        out_shape=jax.ShapeDtypeStruct((n, oh, ow, cout), jnp.bfloat16),
        grid=(2,),
        in_specs=[
            pl.BlockSpec((n, h, wd, c), lambda i: (0, 0, 0, 0)),
            pl.BlockSpec((w.shape[0], tn), lambda i: (0, i)),
            pl.BlockSpec((1, tn), lambda i: (0, i)),
        ],
        out_specs=pl.BlockSpec((n, oh, ow, tn), lambda i: (0, 0, 0, i)),
        compiler_params=pltpu.CompilerParams(
            dimension_semantics=("parallel",),
            vmem_limit_bytes=_vmem_limit(x, w, b)),
    )(x, w, b)


def _conv_hwnc(x, w, b, *, transpose_in=False, split=True):
    if transpose_in:
        n, h, wd, c = x.shape
    else:
        h, wd, n, c = x.shape
    cout = w.shape[1]
    oh, ow = h - 2, wd - 2
    if not split:
        return pl.pallas_call(
            functools.partial(_conv_hwnc_kernel, oh=oh, ow=ow, c=c,
                              transpose_in=transpose_in),
            out_shape=jax.ShapeDtypeStruct((oh, ow, n, cout), jnp.float32),
            in_specs=[pl.BlockSpec(memory_space=pltpu.MemorySpace.VMEM)] * 3,
            out_specs=pl.BlockSpec(memory_space=pltpu.MemorySpace.VMEM),
            compiler_params=pltpu.CompilerParams(
                vmem_limit_bytes=_vmem_limit(x, w, b)),
        )(x, w, b)
    tn = cout // 2
    return pl.pallas_call(
        functools.partial(_conv_hwnc_kernel, oh=oh, ow=ow, c=c,
                          transpose_in=transpose_in),
        out_shape=jax.ShapeDtypeStruct((oh, ow, n, cout), jnp.float32),
        grid=(2,),
        in_specs=[
            pl.BlockSpec(x.shape, lambda i: (0, 0, 0, 0)),
            pl.BlockSpec((w.shape[0], tn), lambda i: (0, i)),
            pl.BlockSpec((1, tn), lambda i: (0, i)),
        ],
        out_specs=pl.BlockSpec((oh, ow, n, tn), lambda i: (0, 0, 0, i)),
        compiler_params=pltpu.CompilerParams(
            dimension_semantics=("parallel",),
            vmem_limit_bytes=_vmem_limit(x, w, b)),
    )(x, w, b)


def _tail(x, w, b, dl_w, dl_b, fc_w, fc_b):
    n, h, wd, c = x.shape
    cout = w.shape[1]
    tn = cout // 2
    nh = dl_w.shape[1]
    h_parts = pl.pallas_call(
        functools.partial(_tail_kernel, c=c),
        out_shape=jax.ShapeDtypeStruct((2, n, nh), jnp.float32),
        grid=(2,),
        in_specs=[
            pl.BlockSpec(x.shape, lambda i: (0, 0, 0, 0)),
            pl.BlockSpec((w.shape[0], tn), lambda i: (0, i)),
            pl.BlockSpec((1, tn), lambda i: (0, i)),
            pl.BlockSpec((tn, nh), lambda i: (i, 0)),
        ],
        out_specs=pl.BlockSpec((1, n, nh), lambda i: (i, 0, 0)),
        compiler_params=pltpu.CompilerParams(
            dimension_semantics=("parallel",),
            vmem_limit_bytes=_vmem_limit(x, w, dl_w)),
    )(x, w, b, dl_w)
    logits = pl.pallas_call(
        _head_kernel,
        out_shape=jax.ShapeDtypeStruct((n, fc_w.shape[1]), jnp.float32),
        in_specs=[pl.BlockSpec(memory_space=pltpu.MemorySpace.VMEM)] * 4,
        out_specs=pl.BlockSpec(memory_space=pltpu.MemorySpace.VMEM),
    )(h_parts, dl_b, fc_w, fc_b)
    return logits


def kernel(x, conv1_w, conv1_b, conv2_w, conv2_b, conv3_w, conv3_b,
           conv4_w, conv4_b, conv5_w, conv5_b, conv6_w, conv6_b,
           conv7_w, conv7_b, conv8_w, conv8_b, dl_w, dl_b, fc_w, fc_b):
    xh = jnp.transpose(x, (0, 2, 3, 1)).astype(jnp.bfloat16)

    h = _conv_single(xh, conv1_w, conv1_b, _conv1_kernel)
    h = _conv_single(h, conv2_w, conv2_b, _conv_batch_kernel)
    h = _conv_single(h, conv3_w, conv3_b, _conv_batch_kernel)
    h = _conv_single(h, conv4_w, conv4_b, _conv_batch_kernel)
    h = _conv_csplit(h, conv5_w, conv5_b)
    h = _conv_csplit(h, conv6_w, conv6_b)
    h = _conv_csplit(h, conv7_w, conv7_b)
    logits = _tail(h, conv8_w, conv8_b, dl_w, dl_b, fc_w, fc_b)
    return logits[:, :2]


# speedup vs baseline: 1.1165x; 1.0012x over previous
import functools

import jax
import jax.numpy as jnp
from jax.experimental import pallas as pl
from jax.experimental.pallas import tpu as pltpu


def _taps_nhwc(x, w_ref, oh, ow, c):
    n = x.shape[0]
    m = n * oh * ow
    a = jnp.concatenate(
        [x[:, kh:kh + oh, kw:kw + ow, :].reshape(m, c)
         for kh in range(3) for kw in range(3)], axis=1)
    return jnp.dot(a, w_ref[...], preferred_element_type=jnp.float32)


def _conv1_kernel(x_ref, w_ref, b_ref, o_ref, *, oh, ow, c):
    n = x_ref.shape[0]
    m = n * oh * ow
    x = x_ref[...]
    kw_pad = w_ref.shape[0]
    pieces = [x[:, kh:kh + oh, kw:kw + ow, :].reshape(m, c)
              for kh in range(3) for kw in range(3)]
    pieces.append(jnp.zeros((m, kw_pad - 9 * c), jnp.bfloat16))
    a = jnp.concatenate(pieces, axis=1)
    r = jnp.dot(a, w_ref[...], preferred_element_type=jnp.float32)
    r = jnp.maximum(r + b_ref[...], 0.0)
    o_ref[...] = r.reshape(n, oh, ow, o_ref.shape[-1]).astype(o_ref.dtype)


def _conv_batch_kernel(x_ref, w_ref, b_ref, o_ref, *, oh, ow, c):
    n = x_ref.shape[0]
    acc = _taps_nhwc(x_ref[...], w_ref, oh, ow, c)
    r = jnp.maximum(acc + b_ref[...], 0.0)
    o_ref[...] = r.reshape(n, oh, ow, o_ref.shape[-1]).astype(o_ref.dtype)


def _tail_kernel(x_ref, w_ref, b_ref, dlw_ref, o_ref, *, c):
    x = x_ref[...]
    n = x.shape[0]
    tn = w_ref.shape[1]
    acc = _taps_nhwc(x, w_ref, 2, 2, c)
    r = jnp.maximum(acc + b_ref[...], 0.0).astype(jnp.bfloat16)
    pooled = jnp.mean(r.reshape(n, 4, tn).astype(jnp.float32), axis=1)
    h_part = jnp.dot(pooled.astype(jnp.bfloat16), dlw_ref[...],
                     preferred_element_type=jnp.float32)
    o_ref[...] = h_part.reshape(o_ref.shape)


def _head_kernel(hp_ref, dlb_ref, fcw_ref, fcb_ref, o_ref):
    h = hp_ref[0] + hp_ref[1] + dlb_ref[...]
    logits = jnp.dot(h.astype(jnp.bfloat16), fcw_ref[...],
                     preferred_element_type=jnp.float32) + fcb_ref[...]
    o_ref[...] = logits


def _vmem_limit(*arrays):
    need = 2 * sum(a.size * a.dtype.itemsize for a in arrays) + (8 << 20)
    return int(min(max(need, 32 << 20), 58 << 20))


def _conv_single(x, w, b, kern_fn):
    n, h, wd, c = x.shape
    cout = w.shape[1]
    oh, ow = h - 2, wd - 2
    return pl.pallas_call(
        functools.partial(kern_fn, oh=oh, ow=ow, c=c),
        out_shape=jax.ShapeDtypeStruct((n, oh, ow, cout), jnp.bfloat16),
        in_specs=[pl.BlockSpec(memory_space=pltpu.MemorySpace.VMEM)] * 3,
        out_specs=pl.BlockSpec(memory_space=pltpu.MemorySpace.VMEM),
        compiler_params=pltpu.CompilerParams(
            vmem_limit_bytes=_vmem_limit(x, w, b)),
    )(x, w, b)


def _conv_csplit(x, w, b):
    n, h, wd, c = x.shape
    cout = w.shape[1]
    oh, ow = h - 2, wd - 2
    tn = cout // 2
    return pl.pallas_call(
        functools.partial(_conv_batch_kernel, oh=oh, ow=ow, c=c),
        out_shape=jax.ShapeDtypeStruct((n, oh, ow, cout), jnp.bfloat16),
        grid=(2,),
        in_specs=[
            pl.BlockSpec((n, h, wd, c), lambda i: (0, 0, 0, 0)),
            pl.BlockSpec((w.shape[0], tn), lambda i: (0, i)),
            pl.BlockSpec((1, tn), lambda i: (0, i)),
        ],
        out_specs=pl.BlockSpec((n, oh, ow, tn), lambda i: (0, 0, 0, i)),
        compiler_params=pltpu.CompilerParams(
            dimension_semantics=("parallel",),
            vmem_limit_bytes=_vmem_limit(x, w, b)),
    )(x, w, b)


def _tail(x, w, b, dl_w, dl_b, fc_w, fc_b):
    n, h, wd, c = x.shape
    cout = w.shape[1]
    tn = cout // 2
    nh = dl_w.shape[1]
    h_parts = pl.pallas_call(
        functools.partial(_tail_kernel, c=c),
        out_shape=jax.ShapeDtypeStruct((2, n, nh), jnp.float32),
        grid=(2,),
        in_specs=[
            pl.BlockSpec(x.shape, lambda i: (0, 0, 0, 0)),
            pl.BlockSpec((w.shape[0], tn), lambda i: (0, i)),
            pl.BlockSpec((1, tn), lambda i: (0, i)),
            pl.BlockSpec((tn, nh), lambda i: (i, 0)),
        ],
        out_specs=pl.BlockSpec((1, n, nh), lambda i: (i, 0, 0)),
        compiler_params=pltpu.CompilerParams(
            dimension_semantics=("parallel",),
            vmem_limit_bytes=_vmem_limit(x, w, dl_w)),
    )(x, w, b, dl_w)
    logits = pl.pallas_call(
        _head_kernel,
        out_shape=jax.ShapeDtypeStruct((n, fc_w.shape[1]), jnp.float32),
        in_specs=[pl.BlockSpec(memory_space=pltpu.MemorySpace.VMEM)] * 4,
        out_specs=pl.BlockSpec(memory_space=pltpu.MemorySpace.VMEM),
    )(h_parts, dl_b, fc_w, fc_b)
    return logits


def kernel(x, conv1_w, conv1_b, conv2_w, conv2_b, conv3_w, conv3_b,
           conv4_w, conv4_b, conv5_w, conv5_b, conv6_w, conv6_b,
           conv7_w, conv7_b, conv8_w, conv8_b, dl_w, dl_b, fc_w, fc_b):
    xh = jnp.transpose(x, (0, 2, 3, 1)).astype(jnp.bfloat16)

    h = _conv_single(xh, conv1_w, conv1_b, _conv1_kernel)
    h = _conv_single(h, conv2_w, conv2_b, _conv_batch_kernel)
    h = _conv_single(h, conv3_w, conv3_b, _conv_batch_kernel)
    h = _conv_single(h, conv4_w, conv4_b, _conv_batch_kernel)
    h = _conv_csplit(h, conv5_w, conv5_b)
    h = _conv_csplit(h, conv6_w, conv6_b)
    h = _conv_csplit(h, conv7_w, conv7_b)
    logits = _tail(h, conv8_w, conv8_b, dl_w, dl_b, fc_w, fc_b)
    return logits[:, :2]
